# baseline (speedup 1.0000x reference)
import functools

import jax
import jax.numpy as jnp
from jax import lax
from jax.experimental import pallas as pl
from jax.experimental.pallas import tpu as pltpu
from jax.experimental.pallas import tpu_sc as plsc

_D = 64
_NC = 2
_NS = 16
_NW = _NC * _NS
_LANES = 16
_W0 = 384


@functools.lru_cache(maxsize=None)
def _build_transpose(V: int):
    n_full = V // _W0

    mesh = plsc.VectorSubcoreMesh(core_axis_name="c", subcore_axis_name="s")
    jg_full = _W0 // _LANES

    @functools.partial(
        pl.kernel,
        mesh=mesh,
        out_type=jax.ShapeDtypeStruct((V * _D,), jnp.float32),
        compiler_params=pltpu.CompilerParams(
            use_tc_tiling_on_sc=True, needs_layout_passes=False),
        scratch_types=[
            pltpu.VMEM((_D, _W0), jnp.float32),
            pltpu.VMEM((_D, _W0), jnp.float32),
            pltpu.VMEM((_W0 * _D,), jnp.float32),
            pltpu.VMEM((_W0 * _D,), jnp.float32),
            pltpu.SemaphoreType.DMA,
            pltpu.SemaphoreType.DMA,
            pltpu.SemaphoreType.DMA,
            pltpu.SemaphoreType.DMA,
        ],
    )
    def transpose_kernel(et_hbm, out_hbm, in0, in1, tb0, tb1,
                         is0, is1, os0, os1):
        wid = lax.axis_index("s") * _NC + lax.axis_index("c")
        inbufs = (in0, in1)
        tbufs = (tb0, tb1)
        isems = (is0, is1)
        osems = (os0, os1)
        iota = lax.iota(jnp.int32, _LANES)
        diag = [(iota + p) & (_LANES - 1) for p in range(_LANES)]
        fvec = [iota * _D + diag[p] for p in range(_LANES)]
        nblk = (n_full - wid + _NW - 1) // _NW

        def in_descs(m, bb):
            c0 = (wid + m * _NW) * _W0
            return [
                pltpu.make_async_copy(
                    et_hbm.at[pl.ds(r * 8, 8), pl.ds(c0, _W0)],
                    inbufs[bb].at[pl.ds(r * 8, 8)],
                    isems[bb])
                for r in range(_D // 8)
            ]

        def out_desc(m, bb):
            c0 = (wid + m * _NW) * _W0
            return pltpu.make_async_copy(
                tbufs[bb], out_hbm.at[pl.ds(c0 * _D, _W0 * _D)],
                osems[bb])

        def transpose_block(bb, n_jg):
            @pl.loop(0, n_jg)
            def _jg(jg):
                jvec = iota + jg * _LANES
                base = (jg * _LANES) * _D
                for dg in range(_D // _LANES):
                    d0 = dg * _LANES
                    src = inbufs[bb].at[pl.ds(d0, _LANES)]
                    dst = tbufs[bb].at[pl.ds(base + d0, _LANES * _D)]
                    for p in range(_LANES):
                        vals = plsc.load_gather(src, [diag[p], jvec])
                        plsc.store_scatter(dst, [fvec[p]], vals)

        @pl.when(nblk > 0)
        def _():
            for d in in_descs(0, 0):
                d.start()

        @pl.when(nblk > 1)
        def _():
            for d in in_descs(1, 1):
                d.start()

        @pl.loop(0, ((nblk + 1) >> 1) << 1, step=2)
        def _pair(t):
            for bb in range(2):
                m = t + bb

                @pl.when(m < nblk)
                def _():
                    for d in in_descs(m, bb):
                        d.wait()

                    @pl.when(m >= 2)
                    def _():
                        out_desc(m - 2, bb).wait()

                    transpose_block(bb, jg_full)
                    out_desc(m, bb).start()

                    @pl.when(m + 2 < nblk)
                    def _():
                        for d in in_descs(m + 2, bb):
                            d.start()

        par = nblk & 1
        for bb in range(2):
            @pl.when((nblk > 1) & (par == bb))
            def _():
                out_desc(nblk - 2, bb).wait()

            @pl.when((nblk > 0) & (par != bb))
            def _():
                out_desc(nblk - 1, bb).wait()

    return transpose_kernel


@functools.lru_cache(maxsize=None)
def _build_gather(B: int, S: int, V2: int):
    b_per_w = B // _NW
    ch = 256
    n_ch = S * (b_per_w // ch)

    mesh = plsc.VectorSubcoreMesh(core_axis_name="c", subcore_axis_name="s")
    n_g = ch // _LANES

    @functools.partial(
        pl.kernel,
        mesh=mesh,
        out_type=jax.ShapeDtypeStruct((S, _D, B), jnp.float32),
        compiler_params=pltpu.CompilerParams(
            use_tc_tiling_on_sc=True, needs_layout_passes=False),
        scratch_types=[
            pltpu.VMEM((S, b_per_w), jnp.int32),
            pltpu.VMEM((2, ch), jnp.int32),
            pltpu.VMEM((2, ch, 2 * _D), jnp.float32),
            pltpu.VMEM((2, _D, ch), jnp.float32),
            pltpu.VMEM((32, 2 * _D), jnp.float32),
            pltpu.SemaphoreType.DMA,
            pltpu.SemaphoreType.DMA,
            pltpu.SemaphoreType.DMA,
            pltpu.SemaphoreType.DMA,
        ],
    )
    def gather_kernel(ids_hbm, table_hbm, tail_hbm, out_hbm,
                      ids_v, idsh_v, gbuf, tbuf, tailv,
                      gs0, gs1, ss0, ss1):
        wid = lax.axis_index("s") * _NC + lax.axis_index("c")
        wb0 = wid * b_per_w
        gsems = (gs0, gs1)
        ssems = (ss0, ss1)
        iota = lax.iota(jnp.int32, _LANES)
        diag = [(iota + p) & (_LANES - 1) for p in range(_LANES)]

        t2 = V2 - 32

        pltpu.sync_copy(ids_hbm.at[:, pl.ds(wb0, b_per_w)], ids_v)
        pltpu.sync_copy(tail_hbm, tailv)

        def halve_ids(k, b):
            s, half = k >> 1, k & 1
            for g in range(n_g):
                ids16 = ids_v[s, pl.ds(half * ch + g * _LANES, _LANES)]
                idsh_v[b, pl.ds(g * _LANES, _LANES)] = ids16 >> 1

        def gather_descs(k, b):
            return [
                pltpu.make_async_copy(
                    table_hbm.at[idsh_v.at[b].at[pl.ds(j * 128, 128)]],
                    gbuf.at[b].at[pl.ds(j * 128, 128)],
                    gsems[b],
                )
                for j in range(ch // 128)
            ]

        def store_desc(k, b):
            s, half = k >> 1, k & 1
            return pltpu.make_async_copy(
                tbuf.at[b],
                out_hbm.at[s, :, pl.ds(wb0 + half * ch, ch)],
                ssems[b],
            )

        def transpose_chunk(k, b):
            s, half = k >> 1, k & 1

            @pl.loop(0, n_g)
            def _g(g):
                rowg = iota + g * _LANES
                ids16 = ids_v[s, pl.ds(half * ch + g * _LANES, _LANES)]
                par64 = (ids16 & 1) << 6
                idsh16 = ids16 >> 1
                tmask = idsh16 >= t2
                anyt = jnp.max(tmask.astype(jnp.int32))

                @pl.when(anyt == 0)
                def _fast():
                    for dg in range(_D // _LANES):
                        d0 = dg * _LANES
                        for p in range(_LANES):
                            dcol = diag[p] + d0
                            vals = plsc.load_gather(
                                gbuf.at[b], [rowg, dcol + par64])
                            plsc.store_scatter(
                                tbuf.at[b], [dcol, rowg], vals)

                @pl.when(anyt != 0)
                def _slow():
                    trow = jnp.maximum(idsh16 - t2, 0)
                    for dg in range(_D // _LANES):
                        d0 = dg * _LANES
                        for p in range(_LANES):
                            dcol = diag[p] + d0
                            vals = plsc.load_gather(
                                gbuf.at[b], [rowg, dcol + par64])
                            tvals = plsc.load_gather(
                                tailv, [trow, dcol + par64])
                            vals = jnp.where(tmask, tvals, vals)
                            plsc.store_scatter(
                                tbuf.at[b], [dcol, rowg], vals)

        def fire_gather(k, b):
            halve_ids(k, b)
            for dsc in gather_descs(k, b):
                dsc.start()

        fire_gather(0, 0)
        fire_gather(1, 1)

        @pl.loop(0, n_ch, step=2)
        def _pair(t):
            for b in range(2):
                k = t + b
                for dsc in gather_descs(k, b):
                    dsc.wait()

                @pl.when(k >= 2)
                def _():
                    store_desc(k - 2, b).wait()

                transpose_chunk(k, b)
                store_desc(k, b).start()

                @pl.when(k + 2 < n_ch)
                def _():
                    fire_gather(k + 2, b)

        store_desc(n_ch - 2, 0).wait()
        store_desc(n_ch - 1, 1).wait()

    return gather_kernel


def kernel(token_ids, embedding):
    b, s = token_ids.shape
    v, d = embedding.shape
    ids_t = token_ids.T.astype(jnp.int32)
    et = embedding.T
    tflat = _build_transpose(v)(et)
    table2 = tflat.reshape(v // 2, 2 * d)
    n_full = v // _W0
    tail2 = embedding[n_full * _W0:].reshape(32, 2 * d)
    out3 = _build_gather(b, s, v // 2)(ids_t, table2, tail2)
    return out3.transpose(2, 0, 1)


# speedup vs baseline: 2.4222x; 2.4222x over previous
import functools

import jax
import jax.numpy as jnp
from jax import lax
from jax.experimental import pallas as pl
from jax.experimental.pallas import tpu as pltpu
from jax.experimental.pallas import tpu_sc as plsc

_D = 64
_NC = 2
_NS = 16
_NW = _NC * _NS
_LANES = 16
_W0 = 384


@functools.lru_cache(maxsize=None)
def _build_transpose(V: int):
    n_full = V // _W0

    mesh = plsc.VectorSubcoreMesh(core_axis_name="c", subcore_axis_name="s")
    jg_full = _W0 // _LANES

    @functools.partial(
        pl.kernel,
        mesh=mesh,
        out_type=jax.ShapeDtypeStruct((V * _D,), jnp.float32),
        compiler_params=pltpu.CompilerParams(
            use_tc_tiling_on_sc=True, needs_layout_passes=False),
        scratch_types=[
            pltpu.VMEM((_D, _W0), jnp.float32),
            pltpu.VMEM((_D, _W0), jnp.float32),
            pltpu.VMEM((_W0 * _D,), jnp.float32),
            pltpu.VMEM((_W0 * _D,), jnp.float32),
            pltpu.SemaphoreType.DMA,
            pltpu.SemaphoreType.DMA,
            pltpu.SemaphoreType.DMA,
            pltpu.SemaphoreType.DMA,
        ],
    )
    def transpose_kernel(et_hbm, out_hbm, in0, in1, tb0, tb1,
                         is0, is1, os0, os1):
        wid = lax.axis_index("s") * _NC + lax.axis_index("c")
        inbufs = (in0, in1)
        tbufs = (tb0, tb1)
        isems = (is0, is1)
        osems = (os0, os1)
        iota = lax.iota(jnp.int32, _LANES)
        diag = [(iota + p) & (_LANES - 1) for p in range(_LANES)]
        fvec = [iota * _D + diag[p] for p in range(_LANES)]
        nblk = (n_full - wid + _NW - 1) // _NW

        def in_descs(m, bb):
            c0 = (wid + m * _NW) * _W0
            return [
                pltpu.make_async_copy(
                    et_hbm.at[pl.ds(r * 8, 8), pl.ds(c0, _W0)],
                    inbufs[bb].at[pl.ds(r * 8, 8)],
                    isems[bb])
                for r in range(_D // 8)
            ]

        def out_desc(m, bb):
            c0 = (wid + m * _NW) * _W0
            return pltpu.make_async_copy(
                tbufs[bb], out_hbm.at[pl.ds(c0 * _D, _W0 * _D)],
                osems[bb])

        def transpose_block(bb, n_jg):
            @pl.loop(0, n_jg)
            def _jg(jg):
                jvec = iota + jg * _LANES
                base = (jg * _LANES) * _D
                for dg in range(_D // _LANES):
                    d0 = dg * _LANES
                    src = inbufs[bb].at[pl.ds(d0, _LANES)]
                    dst = tbufs[bb].at[pl.ds(base + d0, _LANES * _D)]
                    vals = [plsc.load_gather(src, [diag[p], jvec])
                            for p in range(_LANES)]
                    for p in range(_LANES):
                        plsc.store_scatter(dst, [fvec[p]], vals[p])

        @pl.when(nblk > 0)
        def _():
            for d in in_descs(0, 0):
                d.start()

        @pl.when(nblk > 1)
        def _():
            for d in in_descs(1, 1):
                d.start()

        @pl.loop(0, ((nblk + 1) >> 1) << 1, step=2)
        def _pair(t):
            for bb in range(2):
                m = t + bb

                @pl.when(m < nblk)
                def _():
                    for d in in_descs(m, bb):
                        d.wait()

                    @pl.when(m >= 2)
                    def _():
                        out_desc(m - 2, bb).wait()

                    transpose_block(bb, jg_full)
                    out_desc(m, bb).start()

                    @pl.when(m + 2 < nblk)
                    def _():
                        for d in in_descs(m + 2, bb):
                            d.start()

        par = nblk & 1
        for bb in range(2):
            @pl.when((nblk > 1) & (par == bb))
            def _():
                out_desc(nblk - 2, bb).wait()

            @pl.when((nblk > 0) & (par != bb))
            def _():
                out_desc(nblk - 1, bb).wait()

    return transpose_kernel


@functools.lru_cache(maxsize=None)
def _build_gather(B: int, S: int, V2: int):
    b_per_w = B // _NW
    ch = 256
    n_ch = S * (b_per_w // ch)

    mesh = plsc.VectorSubcoreMesh(core_axis_name="c", subcore_axis_name="s")
    n_g = ch // _LANES

    @functools.partial(
        pl.kernel,
        mesh=mesh,
        out_type=jax.ShapeDtypeStruct((S, _D, B), jnp.float32),
        compiler_params=pltpu.CompilerParams(
            use_tc_tiling_on_sc=True, needs_layout_passes=False),
        scratch_types=[
            pltpu.VMEM((S, b_per_w), jnp.int32),
            pltpu.VMEM((2, ch), jnp.int32),
            pltpu.VMEM((2, ch, 2 * _D), jnp.float32),
            pltpu.VMEM((2, _D, ch), jnp.float32),
            pltpu.VMEM((32, 2 * _D), jnp.float32),
            pltpu.SemaphoreType.DMA,
            pltpu.SemaphoreType.DMA,
            pltpu.SemaphoreType.DMA,
            pltpu.SemaphoreType.DMA,
        ],
    )
    def gather_kernel(ids_hbm, table_hbm, tail_hbm, out_hbm,
                      ids_v, idsh_v, gbuf, tbuf, tailv,
                      gs0, gs1, ss0, ss1):
        wid = lax.axis_index("s") * _NC + lax.axis_index("c")
        wb0 = wid * b_per_w
        gsems = (gs0, gs1)
        ssems = (ss0, ss1)
        iota = lax.iota(jnp.int32, _LANES)
        diag = [(iota + p) & (_LANES - 1) for p in range(_LANES)]

        t2 = V2 - 32

        pltpu.sync_copy(ids_hbm.at[:, pl.ds(wb0, b_per_w)], ids_v)
        pltpu.sync_copy(tail_hbm, tailv)

        def halve_ids(k, b):
            s, half = k >> 1, k & 1
            for g in range(n_g):
                ids16 = ids_v[s, pl.ds(half * ch + g * _LANES, _LANES)]
                idsh_v[b, pl.ds(g * _LANES, _LANES)] = ids16 >> 1

        def gather_descs(k, b):
            return [
                pltpu.make_async_copy(
                    table_hbm.at[idsh_v.at[b].at[pl.ds(j * 128, 128)]],
                    gbuf.at[b].at[pl.ds(j * 128, 128)],
                    gsems[b],
                )
                for j in range(ch // 128)
            ]

        def store_desc(k, b):
            s, half = k >> 1, k & 1
            return pltpu.make_async_copy(
                tbuf.at[b],
                out_hbm.at[s, :, pl.ds(wb0 + half * ch, ch)],
                ssems[b],
            )

        def transpose_chunk(k, b):
            s, half = k >> 1, k & 1

            @pl.loop(0, n_g)
            def _g(g):
                rowg = iota + g * _LANES
                ids16 = ids_v[s, pl.ds(half * ch + g * _LANES, _LANES)]
                par64 = (ids16 & 1) << 6
                idsh16 = ids16 >> 1
                tmask = idsh16 >= t2
                anyt = jnp.max(tmask.astype(jnp.int32))

                @pl.when(anyt == 0)
                def _fast():
                    for dg in range(_D // _LANES):
                        d0 = dg * _LANES
                        vals = [plsc.load_gather(
                                    gbuf.at[b], [rowg, diag[p] + d0 + par64])
                                for p in range(_LANES)]
                        for p in range(_LANES):
                            plsc.store_scatter(
                                tbuf.at[b], [diag[p] + d0, rowg], vals[p])

                @pl.when(anyt != 0)
                def _slow():
                    trow = jnp.maximum(idsh16 - t2, 0)
                    for dg in range(_D // _LANES):
                        d0 = dg * _LANES
                        for p in range(_LANES):
                            dcol = diag[p] + d0
                            vals = plsc.load_gather(
                                gbuf.at[b], [rowg, dcol + par64])
                            tvals = plsc.load_gather(
                                tailv, [trow, dcol + par64])
                            vals = jnp.where(tmask, tvals, vals)
                            plsc.store_scatter(
                                tbuf.at[b], [dcol, rowg], vals)

        def fire_gather(k, b):
            halve_ids(k, b)
            for dsc in gather_descs(k, b):
                dsc.start()

        fire_gather(0, 0)
        fire_gather(1, 1)

        @pl.loop(0, n_ch, step=2)
        def _pair(t):
            for b in range(2):
                k = t + b
                for dsc in gather_descs(k, b):
                    dsc.wait()

                @pl.when(k >= 2)
                def _():
                    store_desc(k - 2, b).wait()

                transpose_chunk(k, b)
                store_desc(k, b).start()

                @pl.when(k + 2 < n_ch)
                def _():
                    fire_gather(k + 2, b)

        store_desc(n_ch - 2, 0).wait()
        store_desc(n_ch - 1, 1).wait()

    return gather_kernel


def kernel(token_ids, embedding):
    b, s = token_ids.shape
    v, d = embedding.shape
    ids_t = token_ids.T.astype(jnp.int32)
    et = embedding.T
    tflat = _build_transpose(v)(et)
    table2 = tflat.reshape(v // 2, 2 * d)
    n_full = v // _W0
    tail2 = embedding[n_full * _W0:].reshape(32, 2 * d)
    out3 = _build_gather(b, s, v // 2)(ids_t, table2, tail2)
    return out3.transpose(2, 0, 1)
